# Initial kernel scaffold; baseline (speedup 1.0000x reference)
import jax
import jax.numpy as jnp
from jax.experimental import pallas as pl


def kernel(x, edge_index, edge_attr, We1, be1, W11, b11, W12, b12, We2, be2, W21, b21, W22, b22):
    raise NotImplementedError("write your pallas kernel here")


# speedup vs baseline: 2.4963x; 2.4963x over previous
import functools

import jax
import jax.numpy as jnp
from jax import lax
from jax.experimental import pallas as pl
from jax.experimental.pallas import tpu as pltpu
from jax.experimental.pallas import tpu_sc as plsc

N, E, D, DE, H = 10000, 320000, 128, 16, 128
NC, NS = 2, 16
NW = NC * NS
EPW = E // NW
CH = 80
NCHUNK = EPW // CH
S_ROWS = 624
TAIL0 = NS * S_ROWS
TAIL = N - TAIL0


def _eproj_body(ea_ref, we_ref, be_ref, o_ref):
    ea = ea_ref[...]
    o_ref[...] = lax.dot_general(ea, we_ref[...], (((1,), (0,)), ((), ())),
                                 preferred_element_type=jnp.float32) + be_ref[...]


def _edge_proj(edge_attr, We, be):
    BE = 8000
    return pl.pallas_call(
        _eproj_body,
        grid=(E // BE,),
        in_specs=[
            pl.BlockSpec((BE, DE), lambda i: (i, 0)),
            pl.BlockSpec((DE, D), lambda i: (0, 0)),
            pl.BlockSpec((1, D), lambda i: (0, 0)),
        ],
        out_specs=pl.BlockSpec((BE, D), lambda i: (i, 0)),
        out_shape=jax.ShapeDtypeStruct((E, D), jnp.float32),
    )(edge_attr, We, be.reshape(1, D))


def _mlp_body(h_ref, w1_ref, b1_ref, w2_ref, b2_ref, o_ref):
    h = h_ref[0] + h_ref[1]
    z = jnp.maximum(
        lax.dot_general(h, w1_ref[...], (((1,), (0,)), ((), ())),
                        preferred_element_type=jnp.float32) + b1_ref[...], 0.0)
    o_ref[...] = lax.dot_general(z, w2_ref[...], (((1,), (0,)), ((), ())),
                                 preferred_element_type=jnp.float32) + b2_ref[...]


def _mlp(h2, W1, b1, W2, b2):
    BN = 2000
    return pl.pallas_call(
        _mlp_body,
        grid=(N // BN,),
        in_specs=[
            pl.BlockSpec((NC, BN, D), lambda i: (0, i, 0)),
            pl.BlockSpec((D, H), lambda i: (0, 0)),
            pl.BlockSpec((1, H), lambda i: (0, 0)),
            pl.BlockSpec((H, D), lambda i: (0, 0)),
            pl.BlockSpec((1, D), lambda i: (0, 0)),
        ],
        out_specs=pl.BlockSpec((BN, D), lambda i: (i, 0)),
        out_shape=jax.ShapeDtypeStruct((N, D), jnp.float32),
    )(h2, W1, b1.reshape(1, H), W2, b2.reshape(1, D))


def _sc_body(x_hbm, zero_hbm, src_hbm, dst_hbm, e_hbm, out_hbm,
             sidx, didx, ebuf, xbuf, mbuf, agg):
    c = lax.axis_index("c")
    s = lax.axis_index("s")
    r0 = s * S_ROWS
    @pl.when(c == 0)
    def _seed_x():
        pltpu.sync_copy(x_hbm.at[pl.ds(r0, S_ROWS)], agg.at[pl.ds(r0, S_ROWS)])

        @pl.when(s == 0)
        def _tail_x():
            pltpu.sync_copy(x_hbm.at[pl.ds(TAIL0, TAIL)], agg.at[pl.ds(TAIL0, TAIL)])

    @pl.when(c == 1)
    def _seed_zero():
        pltpu.sync_copy(zero_hbm.at[pl.ds(r0, S_ROWS)], agg.at[pl.ds(r0, S_ROWS)])

        @pl.when(s == 0)
        def _tail_zero():
            pltpu.sync_copy(zero_hbm.at[pl.ds(TAIL0, TAIL)], agg.at[pl.ds(TAIL0, TAIL)])

    plsc.subcore_barrier()

    ebase = (c * NS + s) * EPW

    def chunk(i, carry):
        base = ebase + i * CH
        pltpu.sync_copy(src_hbm.at[pl.ds(base, CH)], sidx)
        pltpu.sync_copy(dst_hbm.at[pl.ds(base, CH)], didx)
        pltpu.sync_copy(e_hbm.at[pl.ds(base, CH)], ebuf)
        pltpu.sync_copy(x_hbm.at[sidx], xbuf)

        def row(j, carry2):
            for k in range(D // 16):
                sl = pl.ds(k * 16, 16)
                mbuf[j, sl] = jnp.maximum(xbuf[j, sl] + ebuf[j, sl], 0.0)
            return carry2

        lax.fori_loop(0, CH, row, 0)
        pltpu.sync_copy(mbuf, agg.at[didx], add=True)
        return carry

    lax.fori_loop(0, NCHUNK, chunk, 0)
    plsc.subcore_barrier()
    pltpu.sync_copy(agg.at[pl.ds(r0, S_ROWS)], out_hbm.at[c, pl.ds(r0, S_ROWS)])

    @pl.when(s == 0)
    def _write_tail():
        pltpu.sync_copy(agg.at[pl.ds(TAIL0, TAIL)], out_hbm.at[c, pl.ds(TAIL0, TAIL)])


@functools.partial(
    pl.kernel,
    out_type=jax.ShapeDtypeStruct((NC, N, D), jnp.float32),
    mesh=plsc.VectorSubcoreMesh(core_axis_name="c", subcore_axis_name="s"),
    scratch_types=[
        pltpu.VMEM((CH,), jnp.int32),
        pltpu.VMEM((CH,), jnp.int32),
        pltpu.VMEM((CH, D), jnp.float32),
        pltpu.VMEM((CH, D), jnp.float32),
        pltpu.VMEM((CH, D), jnp.float32),
        pltpu.VMEM_SHARED((N, D), jnp.float32),
    ],
)
def _sc_layer(x_hbm, zero_hbm, src_hbm, dst_hbm, e_hbm, out_hbm, *scratch):
    _sc_body(x_hbm, zero_hbm, src_hbm, dst_hbm, e_hbm, out_hbm, *scratch)


def kernel(x, edge_index, edge_attr, We1, be1, W11, b11, W12, b12,
           We2, be2, W21, b21, W22, b22):
    src = edge_index[0]
    dst = edge_index[1]
    zero = jnp.zeros((N, D), jnp.float32)
    e1 = _edge_proj(edge_attr, We1, be1)
    h1 = _sc_layer(x, zero, src, dst, e1)
    x2 = _mlp(h1, W11, b11, W12, b12)
    e2 = _edge_proj(edge_attr, We2, be2)
    h2 = _sc_layer(x2, zero, src, dst, e2)
    return _mlp(h2, W21, b21, W22, b22)


# speedup vs baseline: 3.3503x; 1.3421x over previous
import functools

import jax
import jax.numpy as jnp
from jax import lax
from jax.experimental import pallas as pl
from jax.experimental.pallas import tpu as pltpu
from jax.experimental.pallas import tpu_sc as plsc

N, E, D, DE, H = 10000, 320000, 128, 16, 128
NC, NS = 2, 16
NW = NC * NS
EPW = E // NW
CH = 80
NCHUNK = EPW // CH
S_ROWS = 624
TAIL0 = NS * S_ROWS
TAIL = N - TAIL0


def _eproj_body(ea_ref, we_ref, be_ref, o_ref):
    ea = ea_ref[...]
    o_ref[...] = lax.dot_general(ea, we_ref[...], (((1,), (0,)), ((), ())),
                                 preferred_element_type=jnp.float32) + be_ref[...]


def _edge_proj(edge_attr, We, be):
    BE = 8000
    return pl.pallas_call(
        _eproj_body,
        grid=(E // BE,),
        in_specs=[
            pl.BlockSpec((BE, DE), lambda i: (i, 0)),
            pl.BlockSpec((DE, D), lambda i: (0, 0)),
            pl.BlockSpec((1, D), lambda i: (0, 0)),
        ],
        out_specs=pl.BlockSpec((BE, D), lambda i: (i, 0)),
        out_shape=jax.ShapeDtypeStruct((E, D), jnp.float32),
    )(edge_attr, We, be.reshape(1, D))


def _mlp_body(h_ref, w1_ref, b1_ref, w2_ref, b2_ref, o_ref):
    h = h_ref[0] + h_ref[1]
    z = jnp.maximum(
        lax.dot_general(h, w1_ref[...], (((1,), (0,)), ((), ())),
                        preferred_element_type=jnp.float32) + b1_ref[...], 0.0)
    o_ref[...] = lax.dot_general(z, w2_ref[...], (((1,), (0,)), ((), ())),
                                 preferred_element_type=jnp.float32) + b2_ref[...]


def _mlp(h2, W1, b1, W2, b2):
    BN = 2000
    return pl.pallas_call(
        _mlp_body,
        grid=(N // BN,),
        in_specs=[
            pl.BlockSpec((NC, BN, D), lambda i: (0, i, 0)),
            pl.BlockSpec((D, H), lambda i: (0, 0)),
            pl.BlockSpec((1, H), lambda i: (0, 0)),
            pl.BlockSpec((H, D), lambda i: (0, 0)),
            pl.BlockSpec((1, D), lambda i: (0, 0)),
        ],
        out_specs=pl.BlockSpec((BN, D), lambda i: (i, 0)),
        out_shape=jax.ShapeDtypeStruct((N, D), jnp.float32),
    )(h2, W1, b1.reshape(1, H), W2, b2.reshape(1, D))


def _sc_body(x_hbm, zero_hbm, src_hbm, dst_hbm, e_hbm, out_hbm,
             sidx, didx, ebuf, xbuf, mbuf, agg, isem, esem, gsem):
    c = lax.axis_index("c")
    s = lax.axis_index("s")
    r0 = s * S_ROWS
    @pl.when(c == 0)
    def _seed_x():
        pltpu.sync_copy(x_hbm.at[pl.ds(r0, S_ROWS)], agg.at[pl.ds(r0, S_ROWS)])

        @pl.when(s == 0)
        def _tail_x():
            pltpu.sync_copy(x_hbm.at[pl.ds(TAIL0, TAIL)], agg.at[pl.ds(TAIL0, TAIL)])

    @pl.when(c == 1)
    def _seed_zero():
        pltpu.sync_copy(zero_hbm.at[pl.ds(r0, S_ROWS)], agg.at[pl.ds(r0, S_ROWS)])

        @pl.when(s == 0)
        def _tail_zero():
            pltpu.sync_copy(zero_hbm.at[pl.ds(TAIL0, TAIL)], agg.at[pl.ds(TAIL0, TAIL)])

    plsc.subcore_barrier()

    ebase = (c * NS + s) * EPW

    def chunk(i, carry):
        base = ebase + i * CH
        d_src = pltpu.async_copy(src_hbm.at[pl.ds(base, CH)], sidx, isem)
        d_dst = pltpu.async_copy(dst_hbm.at[pl.ds(base, CH)], didx, isem)
        d_e = pltpu.async_copy(e_hbm.at[pl.ds(base, CH)], ebuf, esem)
        d_src.wait()
        d_dst.wait()
        d_g = pltpu.async_copy(x_hbm.at[sidx], xbuf, gsem)
        d_e.wait()
        d_g.wait()

        def row(j, carry2):
            for k in range(D // 16):
                sl = pl.ds(k * 16, 16)
                mbuf[j, sl] = jnp.maximum(xbuf[j, sl] + ebuf[j, sl], 0.0)
            return carry2

        lax.fori_loop(0, CH, row, 0)
        pltpu.sync_copy(mbuf, agg.at[didx], add=True)
        return carry

    lax.fori_loop(0, NCHUNK, chunk, 0)
    plsc.subcore_barrier()
    pltpu.sync_copy(agg.at[pl.ds(r0, S_ROWS)], out_hbm.at[c, pl.ds(r0, S_ROWS)])

    @pl.when(s == 0)
    def _write_tail():
        pltpu.sync_copy(agg.at[pl.ds(TAIL0, TAIL)], out_hbm.at[c, pl.ds(TAIL0, TAIL)])


@functools.partial(
    pl.kernel,
    out_type=jax.ShapeDtypeStruct((NC, N, D), jnp.float32),
    mesh=plsc.VectorSubcoreMesh(core_axis_name="c", subcore_axis_name="s"),
    scratch_types=[
        pltpu.VMEM((CH,), jnp.int32),
        pltpu.VMEM((CH,), jnp.int32),
        pltpu.VMEM((CH, D), jnp.float32),
        pltpu.VMEM((CH, D), jnp.float32),
        pltpu.VMEM((CH, D), jnp.float32),
        pltpu.VMEM_SHARED((N, D), jnp.float32),
        pltpu.SemaphoreType.DMA,
        pltpu.SemaphoreType.DMA,
        pltpu.SemaphoreType.DMA,
    ],
)
def _sc_layer(x_hbm, zero_hbm, src_hbm, dst_hbm, e_hbm, out_hbm, *scratch):
    _sc_body(x_hbm, zero_hbm, src_hbm, dst_hbm, e_hbm, out_hbm, *scratch)


def kernel(x, edge_index, edge_attr, We1, be1, W11, b11, W12, b12,
           We2, be2, W21, b21, W22, b22):
    src = edge_index[0]
    dst = edge_index[1]
    zero = jnp.zeros((N, D), jnp.float32)
    e1 = _edge_proj(edge_attr, We1, be1)
    h1 = _sc_layer(x, zero, src, dst, e1)
    x2 = _mlp(h1, W11, b11, W12, b12)
    e2 = _edge_proj(edge_attr, We2, be2)
    h2 = _sc_layer(x2, zero, src, dst, e2)
    return _mlp(h2, W21, b21, W22, b22)


# speedup vs baseline: 4.9126x; 1.4663x over previous
import functools

import jax
import jax.numpy as jnp
from jax import lax
from jax.experimental import pallas as pl
from jax.experimental.pallas import tpu as pltpu
from jax.experimental.pallas import tpu_sc as plsc

N, E, D, DE, H = 10000, 320000, 128, 16, 128
NC, NS = 2, 16
NW = NC * NS
EPW = E // NW
CH = 40
NCHUNK = EPW // CH
S_ROWS = 624
TAIL0 = NS * S_ROWS
TAIL = N - TAIL0


def _eproj_body(ea_ref, we_ref, be_ref, o_ref):
    ea = ea_ref[...]
    o_ref[...] = lax.dot_general(ea, we_ref[...], (((1,), (0,)), ((), ())),
                                 preferred_element_type=jnp.float32) + be_ref[...]


def _edge_proj(edge_attr, We, be):
    BE = 8000
    return pl.pallas_call(
        _eproj_body,
        grid=(E // BE,),
        in_specs=[
            pl.BlockSpec((BE, DE), lambda i: (i, 0)),
            pl.BlockSpec((DE, D), lambda i: (0, 0)),
            pl.BlockSpec((1, D), lambda i: (0, 0)),
        ],
        out_specs=pl.BlockSpec((BE, D), lambda i: (i, 0)),
        out_shape=jax.ShapeDtypeStruct((E, D), jnp.float32),
    )(edge_attr, We, be.reshape(1, D))


def _mlp_body(h_ref, w1_ref, b1_ref, w2_ref, b2_ref, o_ref):
    h = h_ref[0] + h_ref[1]
    z = jnp.maximum(
        lax.dot_general(h, w1_ref[...], (((1,), (0,)), ((), ())),
                        preferred_element_type=jnp.float32) + b1_ref[...], 0.0)
    o_ref[...] = lax.dot_general(z, w2_ref[...], (((1,), (0,)), ((), ())),
                                 preferred_element_type=jnp.float32) + b2_ref[...]


def _mlp(h2, W1, b1, W2, b2):
    BN = 2000
    return pl.pallas_call(
        _mlp_body,
        grid=(N // BN,),
        in_specs=[
            pl.BlockSpec((NC, BN, D), lambda i: (0, i, 0)),
            pl.BlockSpec((D, H), lambda i: (0, 0)),
            pl.BlockSpec((1, H), lambda i: (0, 0)),
            pl.BlockSpec((H, D), lambda i: (0, 0)),
            pl.BlockSpec((1, D), lambda i: (0, 0)),
        ],
        out_specs=pl.BlockSpec((BN, D), lambda i: (i, 0)),
        out_shape=jax.ShapeDtypeStruct((N, D), jnp.float32),
    )(h2, W1, b1.reshape(1, H), W2, b2.reshape(1, D))


NBUF = 4
NGROUP = NCHUNK // NBUF
NREM = NCHUNK - NGROUP * NBUF


def _sc_body(x_hbm, zero_hbm, src_hbm, dst_hbm, e_hbm, out_hbm, *scratch):
    sidx = scratch[0:4 * NBUF:4]
    didx = scratch[1:4 * NBUF:4]
    ebuf = scratch[2:4 * NBUF:4]
    xbuf = scratch[3:4 * NBUF:4]
    agg = scratch[4 * NBUF]
    sem = scratch[4 * NBUF + 1:]
    c = lax.axis_index("c")
    s = lax.axis_index("s")
    r0 = s * S_ROWS
    @pl.when(c == 0)
    def _seed_x():
        pltpu.sync_copy(x_hbm.at[pl.ds(r0, S_ROWS)], agg.at[pl.ds(r0, S_ROWS)])

        @pl.when(s == 0)
        def _tail_x():
            pltpu.sync_copy(x_hbm.at[pl.ds(TAIL0, TAIL)], agg.at[pl.ds(TAIL0, TAIL)])

    @pl.when(c == 1)
    def _seed_zero():
        pltpu.sync_copy(zero_hbm.at[pl.ds(r0, S_ROWS)], agg.at[pl.ds(r0, S_ROWS)])

        @pl.when(s == 0)
        def _tail_zero():
            pltpu.sync_copy(zero_hbm.at[pl.ds(TAIL0, TAIL)], agg.at[pl.ds(TAIL0, TAIL)])

    plsc.subcore_barrier()

    ebase = (c * NS + s) * EPW

    def start_in(ci, b):
        base = ebase + ci * CH
        pltpu.async_copy(src_hbm.at[pl.ds(base, CH)], sidx[b], sem[b])
        pltpu.async_copy(dst_hbm.at[pl.ds(base, CH)], didx[b], sem[b])
        pltpu.async_copy(e_hbm.at[pl.ds(base, CH)], ebuf[b], sem[b])

    def wait_in(b):
        pltpu.make_async_copy(src_hbm.at[pl.ds(0, CH)], sidx[b], sem[b]).wait()
        pltpu.make_async_copy(dst_hbm.at[pl.ds(0, CH)], didx[b], sem[b]).wait()
        pltpu.make_async_copy(e_hbm.at[pl.ds(0, CH)], ebuf[b], sem[b]).wait()

    def wait_rows(b):
        pltpu.make_async_copy(e_hbm.at[pl.ds(0, CH)], xbuf[b], sem[b]).wait()

    def _maybe(pred, fn):
        if isinstance(pred, bool):
            if pred:
                fn()
        else:
            pl.when(pred)(fn)

    def emit_step(ci, b):
        b1 = (b + 1) % NBUF
        b3 = (b + 3) % NBUF

        def _next_gather():
            wait_in(b1)
            pltpu.async_copy(x_hbm.at[sidx[b1]], xbuf[b1], sem[b1])

        _maybe(ci + 1 < NCHUNK, _next_gather)
        _maybe(ci >= 1, lambda: wait_rows(b3))
        _maybe(ci + 3 < NCHUNK, lambda: start_in(ci + 3, b3))
        wait_rows(b)

        def row(j, carry2):
            for k in range(D // 16):
                sl = pl.ds(k * 16, 16)
                xbuf[b][j, sl] = jnp.maximum(xbuf[b][j, sl] + ebuf[b][j, sl], 0.0)
            return carry2

        lax.fori_loop(0, CH, row, 0)
        pltpu.async_copy(xbuf[b], agg.at[didx[b]], sem[b], add=True)

    start_in(0, 0)
    start_in(1, 1)
    start_in(2, 2)
    wait_in(0)
    pltpu.async_copy(x_hbm.at[sidx[0]], xbuf[0], sem[0])

    def group(g, carry):
        for b in range(NBUF):
            emit_step(g * NBUF + b, b)
        return carry

    lax.fori_loop(0, NGROUP, group, 0)
    for r in range(NREM):
        ci = NGROUP * NBUF + r
        emit_step(ci, ci % NBUF)
    wait_rows((NCHUNK - 1) % NBUF)
    plsc.subcore_barrier()
    pltpu.sync_copy(agg.at[pl.ds(r0, S_ROWS)], out_hbm.at[c, pl.ds(r0, S_ROWS)])

    @pl.when(s == 0)
    def _write_tail():
        pltpu.sync_copy(agg.at[pl.ds(TAIL0, TAIL)], out_hbm.at[c, pl.ds(TAIL0, TAIL)])


@functools.partial(
    pl.kernel,
    out_type=jax.ShapeDtypeStruct((NC, N, D), jnp.float32),
    mesh=plsc.VectorSubcoreMesh(core_axis_name="c", subcore_axis_name="s"),
    scratch_types=(
        [
            t
            for _ in range(NBUF)
            for t in (
                pltpu.VMEM((CH,), jnp.int32),
                pltpu.VMEM((CH,), jnp.int32),
                pltpu.VMEM((CH, D), jnp.float32),
                pltpu.VMEM((CH, D), jnp.float32),
            )
        ]
        + [pltpu.VMEM_SHARED((N, D), jnp.float32)]
        + [pltpu.SemaphoreType.DMA for _ in range(NBUF)]
    ),
)
def _sc_layer(x_hbm, zero_hbm, src_hbm, dst_hbm, e_hbm, out_hbm, *scratch):
    _sc_body(x_hbm, zero_hbm, src_hbm, dst_hbm, e_hbm, out_hbm, *scratch)


def kernel(x, edge_index, edge_attr, We1, be1, W11, b11, W12, b12,
           We2, be2, W21, b21, W22, b22):
    src = edge_index[0]
    dst = edge_index[1]
    zero = jnp.zeros((N, D), jnp.float32)
    e1 = _edge_proj(edge_attr, We1, be1)
    h1 = _sc_layer(x, zero, src, dst, e1)
    x2 = _mlp(h1, W11, b11, W12, b12)
    e2 = _edge_proj(edge_attr, We2, be2)
    h2 = _sc_layer(x2, zero, src, dst, e2)
    return _mlp(h2, W21, b21, W22, b22)
